# Initial kernel scaffold; baseline (speedup 1.0000x reference)
import jax
import jax.numpy as jnp
from jax.experimental import pallas as pl


def kernel(full_features, full_super_indices_10, full_super_indices_21, W1, b1, W2, b2):
    raise NotImplementedError("write your pallas kernel here")


# speedup vs baseline: 1.2498x; 1.2498x over previous
import functools

import jax
import jax.numpy as jnp
import numpy as np
from jax.experimental import pallas as pl

N_POINTS = 320000
D_FEAT = 128
C = 64
N_SP = 4096
N_SP2 = 64
PAD = 128
MASK_RATIO = 0.6

BLK = 1280
NB = N_POINTS // BLK
CHUNK = 512
NDEST = N_SP2 * PAD
NCHUNK = NDEST // CHUNK

_mask_num = int(N_SP * MASK_RATIO)
_perm = np.asarray(jax.random.permutation(jax.random.key(42), N_SP))
_mask_flag_np = np.zeros((N_SP,), dtype=np.int32)
_mask_flag_np[_perm[:_mask_num]] = 1
_MASK_FLAG_ROW = jnp.asarray(_mask_flag_np.reshape(1, N_SP))


def _mlp_segmax_kernel(ids_ref, x_ref, w1_ref, b1_ref, w2_ref, b2_ref, out_ref):
    i = pl.program_id(0)

    @pl.when(i == 0)
    def _init():
        out_ref[...] = jnp.zeros_like(out_ref)

    x = x_ref[...]
    h = jnp.maximum(
        jnp.dot(x, w1_ref[...], preferred_element_type=jnp.float32) + b1_ref[...],
        0.0,
    )
    h = jnp.maximum(
        jnp.dot(h, w2_ref[...], preferred_element_type=jnp.float32) + b2_ref[...],
        0.0,
    )

    ids = ids_ref[0, 0, :]
    ids_col = ids.reshape(BLK, 1)
    s_lo = jnp.min(ids)
    s_hi = jnp.max(ids)

    def body(s, carry):
        m = jnp.max(jnp.where(ids_col == s, h, 0.0), axis=0)
        cur = out_ref[pl.ds(s, 1), :]
        out_ref[pl.ds(s, 1), :] = jnp.maximum(cur, m[None, :])
        return carry

    jax.lax.fori_loop(s_lo, s_hi + 1, body, 0)


def _pad_scatter_kernel(tokens_ref, sp21_ref, flag_ref, out_r_ref, out_m_ref):
    j = pl.program_id(0)
    sp21 = sp21_ref[...]
    flag = flag_ref[...]

    keyc = sp21 * 2 + flag
    e = (jax.lax.broadcasted_iota(jnp.int32, (2 * N_SP2, N_SP), 0) == keyc)
    e = e.astype(jnp.float32)

    s = e
    k = 1
    while k < N_SP:
        s = s + jnp.concatenate(
            [jnp.zeros((2 * N_SP2, k), jnp.float32), s[:, : N_SP - k]], axis=1
        )
        k *= 2
    s_excl = s - e
    pos = jnp.sum(s_excl * e, axis=0, keepdims=True)
    pos = jnp.minimum(pos, float(PAD - 1)).astype(jnp.int32)
    dest = sp21 * PAD + pos

    rows = jax.lax.broadcasted_iota(jnp.int32, (CHUNK, N_SP), 0) + j * CHUNK
    oht = (rows == dest).astype(jnp.float32)

    t = tokens_ref[...]
    wr = (1 - flag).astype(jnp.float32)
    wm = flag.astype(jnp.float32)
    out_r_ref[...] = jnp.dot(oht * wr, t, preferred_element_type=jnp.float32)
    out_m_ref[...] = jnp.dot(oht * wm, t, preferred_element_type=jnp.float32)


@jax.jit
def kernel(full_features, full_super_indices_10, full_super_indices_21, W1, b1, W2, b2):
    ids3 = full_super_indices_10.astype(jnp.int32).reshape(NB, 1, BLK)
    b1r = b1.reshape(1, C)
    b2r = b2.reshape(1, C)

    tokens = pl.pallas_call(
        _mlp_segmax_kernel,
        grid=(NB,),
        in_specs=[
            pl.BlockSpec((1, 1, BLK), lambda i: (i, 0, 0)),
            pl.BlockSpec((BLK, D_FEAT), lambda i: (i, 0)),
            pl.BlockSpec((D_FEAT, C), lambda i: (0, 0)),
            pl.BlockSpec((1, C), lambda i: (0, 0)),
            pl.BlockSpec((C, C), lambda i: (0, 0)),
            pl.BlockSpec((1, C), lambda i: (0, 0)),
        ],
        out_specs=pl.BlockSpec((N_SP, C), lambda i: (0, 0)),
        out_shape=jax.ShapeDtypeStruct((N_SP, C), jnp.float32),
    )(ids3, full_features, W1, b1r, W2, b2r)

    sp21_row = full_super_indices_21.astype(jnp.int32).reshape(1, N_SP)

    out_r, out_m = pl.pallas_call(
        _pad_scatter_kernel,
        grid=(NCHUNK,),
        in_specs=[
            pl.BlockSpec((N_SP, C), lambda j: (0, 0)),
            pl.BlockSpec((1, N_SP), lambda j: (0, 0)),
            pl.BlockSpec((1, N_SP), lambda j: (0, 0)),
        ],
        out_specs=[
            pl.BlockSpec((CHUNK, C), lambda j: (j, 0)),
            pl.BlockSpec((CHUNK, C), lambda j: (j, 0)),
        ],
        out_shape=[
            jax.ShapeDtypeStruct((NDEST, C), jnp.float32),
            jax.ShapeDtypeStruct((NDEST, C), jnp.float32),
        ],
    )(tokens, sp21_row, _MASK_FLAG_ROW)

    return out_r.reshape(N_SP2, PAD, C), out_m.reshape(N_SP2, PAD, C)


# speedup vs baseline: 1.6151x; 1.2922x over previous
import functools

import jax
import jax.numpy as jnp
import numpy as np
from jax.experimental import pallas as pl

N_POINTS = 320000
D_FEAT = 128
C = 64
N_SP = 4096
N_SP2 = 64
PAD = 128
MASK_RATIO = 0.6

BLK = 1280
NB = N_POINTS // BLK
SEG_WIN = 512
NSEG_PAD = N_SP + SEG_WIN
CHUNK = 1024
NDEST = N_SP2 * PAD
NCHUNK = NDEST // CHUNK

_MASK_FLAG_HEX = (
    "ffd90e1fad9f797b73ccd8ea96ecfb22dfecbeefce78d7390c3bab913bcee414"
    "fbfae7ccd67b04b31a6bfcfdf9b5bccdabda2c7b8e93427b3d5f7f7bb487fb1e"
    "affbb15fe9dffb5730de3bd48c5ea8dcb5884bd0ebd09feb711d0fe6bc697461"
    "76defe75f116b9ce1f7e5bcecc1f3c16e111d67f07a367cfd7cf45dbb8f8079b"
    "767946a21f863b3ee4cd66aaabdd2fcfef9dbd1f17ec6ee5ddf07940c9ae9a61"
    "3f8f5c64d8394ef57f1fac6b4f72ebad81cf88dbeb351bfcd43d5d9b966bdb3c"
    "de497ff63795ca374fd35f86d9d78ce7c077a7948757ddbe17f3bd52b11b5635"
    "d94d41cfff8eebdfebb8adf3377aefca6b381fe1dc5fb2b5ff14b4aefe26e301"
    "ff97a625db663313f916faca827c736babea7be0f838ee7e777befcbaddbdade"
    "7ff5b877ccfcce7f45fef944fc27fe1e8757d169516ed8a4fe25d73a15cbfae9"
    "e7439a4be67b995f57ecdeb74ffde2a657f57eea47ebc8fe3e4d39c3a1101acd"
    "d1949fa9397efb78377fc6d5dd9fce27a9bf22173f8f463f7e06fffb5eedbd5f"
    "f566afbcfad849661f5fbebaa2b65f84689bfddff31be8483e1879df836bf168"
    "ddf9d9fab1f75f12d45dfbececfaf2e253275f05e422bfbded94e77594e4dfaa"
    "3db7e8dbf3e8622639f9fcea5e5da3ad80969d9f066acfcf19a5375fb3ff535e"
    "52ae36a54d77d4d7fd63f593dbd6cf9bdd7b70b6e44e5d5eb42cff81d5b3f5a7"
)
_MASK_FLAG_ROW_NP = np.unpackbits(
    np.frombuffer(bytes.fromhex(_MASK_FLAG_HEX), dtype=np.uint8)
).astype(np.int32).reshape(1, N_SP)


def _mlp_segmax_kernel(ids_ref, x_ref, w1_ref, b1_ref, w2_ref, b2_ref, out_ref):
    i = pl.program_id(0)

    @pl.when(i == 0)
    def _init():
        out_ref[...] = jnp.zeros_like(out_ref)

    x = x_ref[...]
    h = jnp.maximum(
        jnp.dot(x, w1_ref[...], preferred_element_type=jnp.float32) + b1_ref[...],
        0.0,
    )
    h = jnp.maximum(
        jnp.dot(h, w2_ref[...], preferred_element_type=jnp.float32) + b2_ref[...],
        0.0,
    )

    ids = ids_ref[0, 0, :]
    ids_row = ids.reshape(1, BLK)
    ids_col = ids.reshape(BLK, 1)
    s_lo = jnp.min(ids)
    s_hi = jnp.max(ids)

    k = 1
    while k < BLK:
        shifted = jnp.concatenate([jnp.zeros((k, C), jnp.float32), h[: BLK - k]], axis=0)
        ids_sh = jnp.concatenate(
            [jnp.full((k, 1), -1, jnp.int32), ids_col[: BLK - k]], axis=0
        )
        h = jnp.where(ids_col == ids_sh, jnp.maximum(h, shifted), h)
        k *= 2

    ids_next = jnp.concatenate(
        [ids_col[1:], jnp.full((1, 1), -1, jnp.int32)], axis=0
    )
    z = jnp.where(ids_col != ids_next, h, 0.0)

    rel = jax.lax.broadcasted_iota(jnp.int32, (SEG_WIN, BLK), 0)
    oht = (rel == (ids_row - s_lo)).astype(jnp.float32)
    part = jnp.dot(oht, z, preferred_element_type=jnp.float32)
    cur = out_ref[pl.ds(s_lo, SEG_WIN), :]
    out_ref[pl.ds(s_lo, SEG_WIN), :] = jnp.maximum(cur, part)

    def body(s, carry):
        m = jnp.max(jnp.where(ids_col == s, h, 0.0), axis=0)
        curr = out_ref[pl.ds(s, 1), :]
        out_ref[pl.ds(s, 1), :] = jnp.maximum(curr, m[None, :])
        return carry

    jax.lax.fori_loop(s_lo + SEG_WIN, s_hi + 1, body, 0)


def _pad_scatter_kernel(tokens_ref, sp21_ref, flag_ref, out_r_ref, out_m_ref):
    j = pl.program_id(0)
    sp21 = sp21_ref[...]
    flag = flag_ref[...]

    keyc = sp21 * 2 + flag
    e = (jax.lax.broadcasted_iota(jnp.int32, (2 * N_SP2, N_SP), 0) == keyc)
    e = e.astype(jnp.float32)

    s = e
    k = 1
    while k < N_SP:
        s = s + jnp.concatenate(
            [jnp.zeros((2 * N_SP2, k), jnp.float32), s[:, : N_SP - k]], axis=1
        )
        k *= 2
    s_excl = s - e
    pos = jnp.sum(s_excl * e, axis=0, keepdims=True)
    pos = jnp.minimum(pos, float(PAD - 1)).astype(jnp.int32)
    dest = sp21 * PAD + pos

    rows = jax.lax.broadcasted_iota(jnp.int32, (CHUNK, N_SP), 0) + j * CHUNK
    oht = (rows == dest).astype(jnp.float32)

    t = tokens_ref[...]
    wr = (1 - flag).astype(jnp.float32)
    wm = flag.astype(jnp.float32)
    out_r_ref[...] = jnp.dot(oht * wr, t, preferred_element_type=jnp.float32)
    out_m_ref[...] = jnp.dot(oht * wm, t, preferred_element_type=jnp.float32)


@jax.jit
def kernel(full_features, full_super_indices_10, full_super_indices_21, W1, b1, W2, b2):
    ids3 = full_super_indices_10.astype(jnp.int32).reshape(NB, 1, BLK)
    b1r = b1.reshape(1, C)
    b2r = b2.reshape(1, C)

    tokens = pl.pallas_call(
        _mlp_segmax_kernel,
        grid=(NB,),
        in_specs=[
            pl.BlockSpec((1, 1, BLK), lambda i: (i, 0, 0)),
            pl.BlockSpec((BLK, D_FEAT), lambda i: (i, 0)),
            pl.BlockSpec((D_FEAT, C), lambda i: (0, 0)),
            pl.BlockSpec((1, C), lambda i: (0, 0)),
            pl.BlockSpec((C, C), lambda i: (0, 0)),
            pl.BlockSpec((1, C), lambda i: (0, 0)),
        ],
        out_specs=pl.BlockSpec((NSEG_PAD, C), lambda i: (0, 0)),
        out_shape=jax.ShapeDtypeStruct((NSEG_PAD, C), jnp.float32),
    )(ids3, full_features, W1, b1r, W2, b2r)
    tokens = tokens[:N_SP]

    sp21_row = full_super_indices_21.astype(jnp.int32).reshape(1, N_SP)

    out_r, out_m = pl.pallas_call(
        _pad_scatter_kernel,
        grid=(NCHUNK,),
        in_specs=[
            pl.BlockSpec((N_SP, C), lambda j: (0, 0)),
            pl.BlockSpec((1, N_SP), lambda j: (0, 0)),
            pl.BlockSpec((1, N_SP), lambda j: (0, 0)),
        ],
        out_specs=[
            pl.BlockSpec((CHUNK, C), lambda j: (j, 0)),
            pl.BlockSpec((CHUNK, C), lambda j: (j, 0)),
        ],
        out_shape=[
            jax.ShapeDtypeStruct((NDEST, C), jnp.float32),
            jax.ShapeDtypeStruct((NDEST, C), jnp.float32),
        ],
    )(tokens, sp21_row, jnp.asarray(_MASK_FLAG_ROW_NP))

    return out_r.reshape(N_SP2, PAD, C), out_m.reshape(N_SP2, PAD, C)


# speedup vs baseline: 2.8579x; 1.7695x over previous
import functools

import jax
import jax.numpy as jnp
import numpy as np
from jax.experimental import pallas as pl

N_POINTS = 320000
D_FEAT = 128
C = 64
N_SP = 4096
N_SP2 = 64
PAD = 128
MASK_RATIO = 0.6

BLK = 1280
NB = N_POINTS // BLK
SEG_WIN = 256
NSEG_PAD = N_SP + SEG_WIN
CHUNK = 1024
NDEST = N_SP2 * PAD
NCHUNK = NDEST // CHUNK

_MASK_FLAG_HEX = (
    "ffd90e1fad9f797b73ccd8ea96ecfb22dfecbeefce78d7390c3bab913bcee414"
    "fbfae7ccd67b04b31a6bfcfdf9b5bccdabda2c7b8e93427b3d5f7f7bb487fb1e"
    "affbb15fe9dffb5730de3bd48c5ea8dcb5884bd0ebd09feb711d0fe6bc697461"
    "76defe75f116b9ce1f7e5bcecc1f3c16e111d67f07a367cfd7cf45dbb8f8079b"
    "767946a21f863b3ee4cd66aaabdd2fcfef9dbd1f17ec6ee5ddf07940c9ae9a61"
    "3f8f5c64d8394ef57f1fac6b4f72ebad81cf88dbeb351bfcd43d5d9b966bdb3c"
    "de497ff63795ca374fd35f86d9d78ce7c077a7948757ddbe17f3bd52b11b5635"
    "d94d41cfff8eebdfebb8adf3377aefca6b381fe1dc5fb2b5ff14b4aefe26e301"
    "ff97a625db663313f916faca827c736babea7be0f838ee7e777befcbaddbdade"
    "7ff5b877ccfcce7f45fef944fc27fe1e8757d169516ed8a4fe25d73a15cbfae9"
    "e7439a4be67b995f57ecdeb74ffde2a657f57eea47ebc8fe3e4d39c3a1101acd"
    "d1949fa9397efb78377fc6d5dd9fce27a9bf22173f8f463f7e06fffb5eedbd5f"
    "f566afbcfad849661f5fbebaa2b65f84689bfddff31be8483e1879df836bf168"
    "ddf9d9fab1f75f12d45dfbececfaf2e253275f05e422bfbded94e77594e4dfaa"
    "3db7e8dbf3e8622639f9fcea5e5da3ad80969d9f066acfcf19a5375fb3ff535e"
    "52ae36a54d77d4d7fd63f593dbd6cf9bdd7b70b6e44e5d5eb42cff81d5b3f5a7"
)
_MASK_FLAG_ROW_NP = np.unpackbits(
    np.frombuffer(bytes.fromhex(_MASK_FLAG_HEX), dtype=np.uint8)
).astype(np.int32).reshape(1, N_SP)


def _mlp_segmax_kernel(ids_ref, x_ref, w1_ref, b1_ref, w2_ref, b2_ref, out_ref):
    i = pl.program_id(0)

    @pl.when(i == 0)
    def _init():
        out_ref[...] = jnp.zeros_like(out_ref)

    x = x_ref[...]
    h1 = jnp.maximum(
        jax.lax.dot_general(
            w1_ref[...], x, (((0,), (1,)), ((), ())),
            preferred_element_type=jnp.float32,
        ) + b1_ref[...],
        0.0,
    )
    h = jnp.maximum(
        jax.lax.dot_general(
            w2_ref[...], h1, (((0,), (0,)), ((), ())),
            preferred_element_type=jnp.float32,
        ) + b2_ref[...],
        0.0,
    )

    ids = ids_ref[0, 0, :]
    ids_row = ids.reshape(1, BLK)
    s_lo = jnp.min(ids)
    s_hi = jnp.max(ids)

    k = 1
    while k < BLK:
        shifted = jnp.concatenate(
            [jnp.zeros((C, k), jnp.float32), h[:, : BLK - k]], axis=1
        )
        ids_sh = jnp.concatenate(
            [jnp.full((1, k), -1, jnp.int32), ids_row[:, : BLK - k]], axis=1
        )
        h = jnp.where(ids_row == ids_sh, jnp.maximum(h, shifted), h)
        k *= 2

    ids_next = jnp.concatenate(
        [ids_row[:, 1:], jnp.full((1, 1), -1, jnp.int32)], axis=1
    )
    z = jnp.where(ids_row != ids_next, h, 0.0)

    rel = jax.lax.broadcasted_iota(jnp.int32, (SEG_WIN, BLK), 0)
    oht = (rel == (ids_row - s_lo)).astype(jnp.float32)
    part = jax.lax.dot_general(
        oht, z, (((1,), (1,)), ((), ())), preferred_element_type=jnp.float32
    )
    cur = out_ref[pl.ds(s_lo, SEG_WIN), :]
    out_ref[pl.ds(s_lo, SEG_WIN), :] = jnp.maximum(cur, part)

    def body(s, carry):
        m = jnp.max(jnp.where(ids_row == s, h, 0.0), axis=1)
        curr = out_ref[pl.ds(s, 1), :]
        out_ref[pl.ds(s, 1), :] = jnp.maximum(curr, m[None, :])
        return carry

    jax.lax.fori_loop(s_lo + SEG_WIN, s_hi + 1, body, 0)


def _pad_scatter_kernel(tokens_ref, sp21_ref, flag_ref, out_r_ref, out_m_ref):
    j = pl.program_id(0)
    sp21 = sp21_ref[...]
    flag = flag_ref[...]

    keyc = sp21 * 2 + flag
    e = (jax.lax.broadcasted_iota(jnp.int32, (2 * N_SP2, N_SP), 0) == keyc)
    e = e.astype(jnp.float32)

    s = e
    k = 1
    while k < N_SP:
        s = s + jnp.concatenate(
            [jnp.zeros((2 * N_SP2, k), jnp.float32), s[:, : N_SP - k]], axis=1
        )
        k *= 2
    s_excl = s - e
    pos = jnp.sum(s_excl * e, axis=0, keepdims=True)
    pos = jnp.minimum(pos, float(PAD - 1)).astype(jnp.int32)
    dest = sp21 * PAD + pos

    rows = jax.lax.broadcasted_iota(jnp.int32, (CHUNK, N_SP), 0) + j * CHUNK
    oht = (rows == dest).astype(jnp.float32)

    t = tokens_ref[...]
    wr = (1 - flag).astype(jnp.float32)
    wm = flag.astype(jnp.float32)
    out_r_ref[...] = jnp.dot(oht * wr, t, preferred_element_type=jnp.float32)
    out_m_ref[...] = jnp.dot(oht * wm, t, preferred_element_type=jnp.float32)


@jax.jit
def kernel(full_features, full_super_indices_10, full_super_indices_21, W1, b1, W2, b2):
    ids3 = full_super_indices_10.astype(jnp.int32).reshape(NB, 1, BLK)
    b1r = b1.reshape(C, 1)
    b2r = b2.reshape(C, 1)

    tokens = pl.pallas_call(
        _mlp_segmax_kernel,
        grid=(NB,),
        in_specs=[
            pl.BlockSpec((1, 1, BLK), lambda i: (i, 0, 0)),
            pl.BlockSpec((BLK, D_FEAT), lambda i: (i, 0)),
            pl.BlockSpec((D_FEAT, C), lambda i: (0, 0)),
            pl.BlockSpec((C, 1), lambda i: (0, 0)),
            pl.BlockSpec((C, C), lambda i: (0, 0)),
            pl.BlockSpec((C, 1), lambda i: (0, 0)),
        ],
        out_specs=pl.BlockSpec((NSEG_PAD, C), lambda i: (0, 0)),
        out_shape=jax.ShapeDtypeStruct((NSEG_PAD, C), jnp.float32),
    )(ids3, full_features, W1, b1r, W2, b2r)
    tokens = tokens[:N_SP]

    sp21_row = full_super_indices_21.astype(jnp.int32).reshape(1, N_SP)

    out_r, out_m = pl.pallas_call(
        _pad_scatter_kernel,
        grid=(NCHUNK,),
        in_specs=[
            pl.BlockSpec((N_SP, C), lambda j: (0, 0)),
            pl.BlockSpec((1, N_SP), lambda j: (0, 0)),
            pl.BlockSpec((1, N_SP), lambda j: (0, 0)),
        ],
        out_specs=[
            pl.BlockSpec((CHUNK, C), lambda j: (j, 0)),
            pl.BlockSpec((CHUNK, C), lambda j: (j, 0)),
        ],
        out_shape=[
            jax.ShapeDtypeStruct((NDEST, C), jnp.float32),
            jax.ShapeDtypeStruct((NDEST, C), jnp.float32),
        ],
    )(tokens, sp21_row, jnp.asarray(_MASK_FLAG_ROW_NP))

    return out_r.reshape(N_SP2, PAD, C), out_m.reshape(N_SP2, PAD, C)


# speedup vs baseline: 3.0625x; 1.0716x over previous
import functools

import jax
import jax.numpy as jnp
import numpy as np
from jax.experimental import pallas as pl

N_POINTS = 320000
D_FEAT = 128
C = 64
N_SP = 4096
N_SP2 = 64
PAD = 128
MASK_RATIO = 0.6

BLK = 2560
NB = N_POINTS // BLK
SEG_WIN = 256
NSEG_PAD = N_SP + SEG_WIN
CHUNK = 1024
NDEST = N_SP2 * PAD
NCHUNK = NDEST // CHUNK

_MASK_FLAG_HEX = (
    "ffd90e1fad9f797b73ccd8ea96ecfb22dfecbeefce78d7390c3bab913bcee414"
    "fbfae7ccd67b04b31a6bfcfdf9b5bccdabda2c7b8e93427b3d5f7f7bb487fb1e"
    "affbb15fe9dffb5730de3bd48c5ea8dcb5884bd0ebd09feb711d0fe6bc697461"
    "76defe75f116b9ce1f7e5bcecc1f3c16e111d67f07a367cfd7cf45dbb8f8079b"
    "767946a21f863b3ee4cd66aaabdd2fcfef9dbd1f17ec6ee5ddf07940c9ae9a61"
    "3f8f5c64d8394ef57f1fac6b4f72ebad81cf88dbeb351bfcd43d5d9b966bdb3c"
    "de497ff63795ca374fd35f86d9d78ce7c077a7948757ddbe17f3bd52b11b5635"
    "d94d41cfff8eebdfebb8adf3377aefca6b381fe1dc5fb2b5ff14b4aefe26e301"
    "ff97a625db663313f916faca827c736babea7be0f838ee7e777befcbaddbdade"
    "7ff5b877ccfcce7f45fef944fc27fe1e8757d169516ed8a4fe25d73a15cbfae9"
    "e7439a4be67b995f57ecdeb74ffde2a657f57eea47ebc8fe3e4d39c3a1101acd"
    "d1949fa9397efb78377fc6d5dd9fce27a9bf22173f8f463f7e06fffb5eedbd5f"
    "f566afbcfad849661f5fbebaa2b65f84689bfddff31be8483e1879df836bf168"
    "ddf9d9fab1f75f12d45dfbececfaf2e253275f05e422bfbded94e77594e4dfaa"
    "3db7e8dbf3e8622639f9fcea5e5da3ad80969d9f066acfcf19a5375fb3ff535e"
    "52ae36a54d77d4d7fd63f593dbd6cf9bdd7b70b6e44e5d5eb42cff81d5b3f5a7"
)
_MASK_FLAG_ROW_NP = np.unpackbits(
    np.frombuffer(bytes.fromhex(_MASK_FLAG_HEX), dtype=np.uint8)
).astype(np.int32).reshape(1, N_SP)


def _mlp_segmax_kernel(ids_ref, x_ref, w1_ref, b1_ref, w2_ref, b2_ref, out_ref):
    i = pl.program_id(0)

    @pl.when(i == 0)
    def _init():
        out_ref[...] = jnp.zeros_like(out_ref)

    x = x_ref[...]
    h1 = jnp.maximum(
        jax.lax.dot_general(
            w1_ref[...], x, (((0,), (1,)), ((), ())),
            preferred_element_type=jnp.float32,
        ) + b1_ref[...],
        0.0,
    )
    h = jnp.maximum(
        jax.lax.dot_general(
            w2_ref[...], h1, (((0,), (0,)), ((), ())),
            preferred_element_type=jnp.float32,
        ) + b2_ref[...],
        0.0,
    )

    ids = ids_ref[0, 0, :]
    ids_row = ids.reshape(1, BLK)
    s_lo = jnp.min(ids)
    s_hi = jnp.max(ids)

    k = 1
    while k < BLK:
        shifted = jnp.concatenate(
            [jnp.zeros((C, k), jnp.float32), h[:, : BLK - k]], axis=1
        )
        ids_sh = jnp.concatenate(
            [jnp.full((1, k), -1, jnp.int32), ids_row[:, : BLK - k]], axis=1
        )
        h = jnp.where(ids_row == ids_sh, jnp.maximum(h, shifted), h)
        k *= 2

    ids_next = jnp.concatenate(
        [ids_row[:, 1:], jnp.full((1, 1), -1, jnp.int32)], axis=1
    )
    z = jnp.where(ids_row != ids_next, h, 0.0)

    rel = jax.lax.broadcasted_iota(jnp.int32, (SEG_WIN, BLK), 0)
    oht = (rel == (ids_row - s_lo)).astype(jnp.float32)
    part = jax.lax.dot_general(
        oht, z, (((1,), (1,)), ((), ())), preferred_element_type=jnp.float32
    )
    cur = out_ref[pl.ds(s_lo, SEG_WIN), :]
    out_ref[pl.ds(s_lo, SEG_WIN), :] = jnp.maximum(cur, part)

    def body(s, carry):
        m = jnp.max(jnp.where(ids_row == s, h, 0.0), axis=1)
        curr = out_ref[pl.ds(s, 1), :]
        out_ref[pl.ds(s, 1), :] = jnp.maximum(curr, m[None, :])
        return carry

    jax.lax.fori_loop(s_lo + SEG_WIN, s_hi + 1, body, 0)


def _pad_scatter_kernel(tokens_ref, sp21_ref, flag_ref, out_r_ref, out_m_ref):
    j = pl.program_id(0)
    sp21 = sp21_ref[...]
    flag = flag_ref[...]

    keyc = sp21 * 2 + flag
    e = (jax.lax.broadcasted_iota(jnp.int32, (2 * N_SP2, N_SP), 0) == keyc)
    e = e.astype(jnp.float32)

    s = e
    k = 1
    while k < N_SP:
        s = s + jnp.concatenate(
            [jnp.zeros((2 * N_SP2, k), jnp.float32), s[:, : N_SP - k]], axis=1
        )
        k *= 2
    s_excl = s - e
    pos = jnp.sum(s_excl * e, axis=0, keepdims=True)
    pos = jnp.minimum(pos, float(PAD - 1)).astype(jnp.int32)
    dest = sp21 * PAD + pos

    rows = jax.lax.broadcasted_iota(jnp.int32, (CHUNK, N_SP), 0) + j * CHUNK
    oht = (rows == dest).astype(jnp.float32)

    t = tokens_ref[...]
    wr = (1 - flag).astype(jnp.float32)
    wm = flag.astype(jnp.float32)
    out_r_ref[...] = jnp.dot(oht * wr, t, preferred_element_type=jnp.float32)
    out_m_ref[...] = jnp.dot(oht * wm, t, preferred_element_type=jnp.float32)


@jax.jit
def kernel(full_features, full_super_indices_10, full_super_indices_21, W1, b1, W2, b2):
    ids3 = full_super_indices_10.astype(jnp.int32).reshape(NB, 1, BLK)
    b1r = b1.reshape(C, 1)
    b2r = b2.reshape(C, 1)

    tokens = pl.pallas_call(
        _mlp_segmax_kernel,
        grid=(NB,),
        in_specs=[
            pl.BlockSpec((1, 1, BLK), lambda i: (i, 0, 0)),
            pl.BlockSpec((BLK, D_FEAT), lambda i: (i, 0)),
            pl.BlockSpec((D_FEAT, C), lambda i: (0, 0)),
            pl.BlockSpec((C, 1), lambda i: (0, 0)),
            pl.BlockSpec((C, C), lambda i: (0, 0)),
            pl.BlockSpec((C, 1), lambda i: (0, 0)),
        ],
        out_specs=pl.BlockSpec((NSEG_PAD, C), lambda i: (0, 0)),
        out_shape=jax.ShapeDtypeStruct((NSEG_PAD, C), jnp.float32),
    )(ids3, full_features, W1, b1r, W2, b2r)
    tokens = tokens[:N_SP]

    sp21_row = full_super_indices_21.astype(jnp.int32).reshape(1, N_SP)

    out_r, out_m = pl.pallas_call(
        _pad_scatter_kernel,
        grid=(NCHUNK,),
        in_specs=[
            pl.BlockSpec((N_SP, C), lambda j: (0, 0)),
            pl.BlockSpec((1, N_SP), lambda j: (0, 0)),
            pl.BlockSpec((1, N_SP), lambda j: (0, 0)),
        ],
        out_specs=[
            pl.BlockSpec((CHUNK, C), lambda j: (j, 0)),
            pl.BlockSpec((CHUNK, C), lambda j: (j, 0)),
        ],
        out_shape=[
            jax.ShapeDtypeStruct((NDEST, C), jnp.float32),
            jax.ShapeDtypeStruct((NDEST, C), jnp.float32),
        ],
    )(tokens, sp21_row, jnp.asarray(_MASK_FLAG_ROW_NP))

    return out_r.reshape(N_SP2, PAD, C), out_m.reshape(N_SP2, PAD, C)
